# baseline (speedup 1.0000x reference)
import jax
import jax.numpy as jnp
from jax import lax
from jax.experimental import pallas as pl
from jax.experimental.pallas import tpu as pltpu
from jax.experimental.pallas import tpu_sc as plsc

EMB_DIM = 64
NUM_CORES = 2
NUM_SUBCORES = 16
NUM_WORKERS = NUM_CORES * NUM_SUBCORES
CHUNK = 128
NBUF = 4


def _sc_body(words_hbm, table_hbm, out_hbm, idx_v, rows_v, g_sem, s_sem):
    n_chunks = words_hbm.shape[1]
    n_groups = n_chunks // NBUF
    wid = lax.axis_index("s") * NUM_CORES + lax.axis_index("c")
    pltpu.sync_copy(words_hbm.at[wid], idx_v)

    def start_gather(j, b):
        pltpu.async_copy(table_hbm.at[idx_v.at[j]], rows_v.at[b], g_sem.at[b])

    def wait_gather(j, b):
        pltpu.make_async_copy(
            table_hbm.at[idx_v.at[j]], rows_v.at[b], g_sem.at[b]
        ).wait()

    def start_store(j, b):
        pltpu.async_copy(rows_v.at[b], out_hbm.at[wid, j], s_sem.at[b])

    def wait_store(j, b):
        pltpu.make_async_copy(
            rows_v.at[b], out_hbm.at[wid, j], s_sem.at[b]
        ).wait()

    for b in range(NBUF):
        start_gather(b, b)

    def outer(g, carry):
        j0 = g * NBUF
        for b in range(NBUF):
            wait_gather(j0 + b, b)
            start_store(j0 + b, b)
        for b in range(NBUF):
            wait_store(j0 + b, b)

            @pl.when(g < n_groups - 1)
            def _():
                start_gather(j0 + b + NBUF, b)

        return carry

    lax.fori_loop(0, n_groups, outer, 0)


def kernel(words, weight):
    b, s = words.shape
    total = b * s
    n_chunks = total // (NUM_WORKERS * CHUNK)
    words_r = words.reshape(NUM_WORKERS, n_chunks, CHUNK)

    mesh = plsc.VectorSubcoreMesh(core_axis_name="c", subcore_axis_name="s")
    out = pl.kernel(
        _sc_body,
        out_type=jax.ShapeDtypeStruct(
            (NUM_WORKERS, n_chunks, CHUNK, EMB_DIM), jnp.float32
        ),
        mesh=mesh,
        compiler_params=pltpu.CompilerParams(use_tc_tiling_on_sc=False),
        scratch_types=[
            pltpu.VMEM((n_chunks, CHUNK), jnp.int32),
            pltpu.VMEM((NBUF, CHUNK, EMB_DIM), jnp.float32),
            pltpu.SemaphoreType.DMA((NBUF,)),
            pltpu.SemaphoreType.DMA((NBUF,)),
        ],
    )(words_r, weight)
    return out.reshape(b, s, EMB_DIM)
